# Initial kernel scaffold; baseline (speedup 1.0000x reference)
import jax
import jax.numpy as jnp
from jax.experimental import pallas as pl


def kernel(p0, f0, params):
    raise NotImplementedError("write your pallas kernel here")


# speedup vs baseline: 4.9981x; 4.9981x over previous
import functools

import jax
import jax.numpy as jnp
from jax import lax
from jax.experimental import pallas as pl
from jax.experimental.pallas import tpu as pltpu
from jax.experimental.pallas import tpu_sc as plsc

_K = 32
_NW = 32
_EPS = 1e-5


def _stem_matmul(x, wT, b):
    R, Cin = x.shape
    Cout = wT.shape[1]
    Nb = min(512, R)

    def body(x_ref, w_ref, b_ref, o_ref):
        o_ref[...] = (
            jnp.dot(x_ref[...], w_ref[...], preferred_element_type=jnp.float32)
            + b_ref[...]
        )

    return pl.pallas_call(
        body,
        grid=(R // Nb,),
        in_specs=[
            pl.BlockSpec((Nb, Cin), lambda i: (i, 0)),
            pl.BlockSpec((Cin, Cout), lambda i: (0, 0)),
            pl.BlockSpec((1, Cout), lambda i: (0, 0)),
        ],
        out_specs=pl.BlockSpec((Nb, Cout), lambda i: (i, 0)),
        out_shape=jax.ShapeDtypeStruct((R, Cout), jnp.float32),
    )(x, wT, b)


def _knn(q_rows, sT, r2):
    B, M, _ = q_rows.shape
    N = sT.shape[2]
    Mb = min(128, M)

    def body(q_ref, s_ref, o_ref, d2_ref):
        qx = q_ref[0, :, 0:1]
        qy = q_ref[0, :, 1:2]
        qz = q_ref[0, :, 2:3]
        sx = s_ref[0, 0:1, :]
        sy = s_ref[0, 1:2, :]
        sz = s_ref[0, 2:3, :]
        dx = qx - sx
        dy = qy - sy
        dz = qz - sz
        d2_ref[...] = dx * dx + dy * dy + dz * dz

        iota_n = lax.broadcasted_iota(jnp.int32, (Mb, N), 1)
        kio = lax.broadcasted_iota(jnp.int32, (Mb, _K), 1)

        def step(k, carry):
            dl, il, acc_d, acc_i = carry
            d2v = d2_ref[...]
            later = (d2v > dl) | ((d2v == dl) & (iota_n > il))
            cand = jnp.where(later, d2v, jnp.inf)
            m = jnp.min(cand, axis=1, keepdims=True)
            selc = jnp.where(cand == m, iota_n, N)
            sel = jnp.min(selc, axis=1, keepdims=True)
            acc_d = jnp.where(kio == k, m, acc_d)
            acc_i = jnp.where(kio == k, sel, acc_i)
            return m, sel, acc_d, acc_i

        init = (
            jnp.full((Mb, 1), -jnp.inf, jnp.float32),
            jnp.full((Mb, 1), -1, jnp.int32),
            jnp.zeros((Mb, _K), jnp.float32),
            jnp.zeros((Mb, _K), jnp.int32),
        )
        _, _, acc_d, acc_i = lax.fori_loop(0, _K, step, init)
        idx0 = acc_i[:, 0:1]
        idxc = jnp.where(acc_d > r2, idx0, acc_i)
        b = pl.program_id(0)
        o_ref[0] = idxc + b * N

    return pl.pallas_call(
        body,
        grid=(B, M // Mb),
        in_specs=[
            pl.BlockSpec((1, Mb, 3), lambda b, j: (b, j, 0)),
            pl.BlockSpec((1, 3, N), lambda b, j: (b, 0, 0)),
        ],
        out_specs=pl.BlockSpec((1, Mb, _K), lambda b, j: (b, j, 0)),
        out_shape=jax.ShapeDtypeStruct((B, M, _K), jnp.int32),
        scratch_shapes=[pltpu.VMEM((Mb, N), jnp.float32)],
    )(q_rows, sT)


def _gather_rows(table, idx, qpos, Q, Wt):
    qpw = Q // _NW
    nidx = qpw * _K
    chunk_i = min(128, nidx)
    nch = nidx // chunk_i

    mesh = plsc.VectorSubcoreMesh(
        core_axis_name="c", subcore_axis_name="s", num_cores=2, num_subcores=16
    )

    @functools.partial(
        pl.kernel,
        out_type=jax.ShapeDtypeStruct((Q * _K, Wt), jnp.float32),
        mesh=mesh,
        scratch_types=[
            pltpu.VMEM((nidx,), jnp.int32),
            pltpu.VMEM((qpw, 16), jnp.float32),
            pltpu.VMEM((chunk_i, Wt), jnp.float32),
            pltpu.SemaphoreType.DMA,
        ],
    )
    def sc_kernel(t_hbm, idx_hbm, qp_hbm, out_hbm, idx_v, qp_v, rows_v, sem):
        wid = lax.axis_index("s") * 2 + lax.axis_index("c")
        pltpu.sync_copy(idx_hbm.at[pl.ds(wid * nidx, nidx)], idx_v)
        pltpu.sync_copy(qp_hbm.at[pl.ds(wid * qpw, qpw)], qp_v)
        base = wid * nidx

        def chunk(ch, _):
            pltpu.async_copy(
                t_hbm.at[idx_v.at[pl.ds(ch * chunk_i, chunk_i)]], rows_v, sem
            ).wait()

            def rfix(r, _):
                q = ch * (chunk_i // _K) + r // _K
                rows_v[r, pl.ds(0, 16)] = (
                    rows_v[r, pl.ds(0, 16)] - qp_v[q, :]
                )
                return 0

            lax.fori_loop(0, chunk_i, rfix, 0)
            pltpu.sync_copy(rows_v, out_hbm.at[pl.ds(base + ch * chunk_i,
                                                     chunk_i)])
            return 0

        lax.fori_loop(0, nch, chunk, 0)

    return sc_kernel(table, idx, qpos)


def _matmul(x, wT):
    R, Cin = x.shape
    Cout = wT.shape[1]
    Nb = min(512, R)

    def body(x_ref, w_ref, o_ref):
        o_ref[...] = jnp.dot(x_ref[...], w_ref[...],
                             preferred_element_type=jnp.float32)

    return pl.pallas_call(
        body,
        grid=(R // Nb,),
        in_specs=[
            pl.BlockSpec((Nb, Cin), lambda i: (i, 0)),
            pl.BlockSpec((Cin, Cout), lambda i: (0, 0)),
        ],
        out_specs=pl.BlockSpec((Nb, Cout), lambda i: (i, 0)),
        out_shape=jax.ShapeDtypeStruct((R, Cout), jnp.float32),
    )(x, wT)


def _colreduce(y, mean=None):
    R, C = y.shape
    Nb = min(512, R)

    def body(y_ref, m_ref, o_ref):
        i = pl.program_id(0)

        @pl.when(i == 0)
        def _():
            o_ref[...] = jnp.zeros_like(o_ref)

        d = y_ref[...] - m_ref[...]
        o_ref[...] += jnp.sum(d * d, axis=0, keepdims=True)

    def body_plain(y_ref, o_ref):
        i = pl.program_id(0)

        @pl.when(i == 0)
        def _():
            o_ref[...] = jnp.zeros_like(o_ref)

        o_ref[...] += jnp.sum(y_ref[...], axis=0, keepdims=True)

    if mean is None:
        return pl.pallas_call(
            body_plain,
            grid=(R // Nb,),
            in_specs=[pl.BlockSpec((Nb, C), lambda i: (i, 0))],
            out_specs=pl.BlockSpec((1, C), lambda i: (0, 0)),
            out_shape=jax.ShapeDtypeStruct((1, C), jnp.float32),
        )(y)
    return pl.pallas_call(
        body,
        grid=(R // Nb,),
        in_specs=[pl.BlockSpec((Nb, C), lambda i: (i, 0)),
                  pl.BlockSpec((1, C), lambda i: (0, 0))],
        out_specs=pl.BlockSpec((1, C), lambda i: (0, 0)),
        out_shape=jax.ShapeDtypeStruct((1, C), jnp.float32),
    )(y, mean)


def _bn_relu_maxpool(y3, s1, s2, gamma, beta, P):
    Q, K, C = y3.shape
    Qb = min(256, Q)

    def body(y_ref, s1_ref, s2_ref, g_ref, b_ref, o_ref):
        mean = s1_ref[...] / P
        var = s2_ref[...] / P
        yv = y_ref[...]
        bn = ((yv - mean[None]) / jnp.sqrt(var[None] + _EPS)
              * g_ref[...][None] + b_ref[...][None])
        o_ref[...] = jnp.max(jnp.maximum(bn, 0.0), axis=1)

    return pl.pallas_call(
        body,
        grid=(Q // Qb,),
        in_specs=[
            pl.BlockSpec((Qb, K, C), lambda i: (i, 0, 0)),
            pl.BlockSpec((1, C), lambda i: (0, 0)),
            pl.BlockSpec((1, C), lambda i: (0, 0)),
            pl.BlockSpec((1, C), lambda i: (0, 0)),
            pl.BlockSpec((1, C), lambda i: (0, 0)),
        ],
        out_specs=pl.BlockSpec((Qb, C), lambda i: (i, 0)),
        out_shape=jax.ShapeDtypeStruct((Q, C), jnp.float32),
    )(y3, s1, s2, gamma, beta)


def _pw_block(y, identity, w1T, g1, b1, w2T, g2, b2):
    Q, C = y.shape

    def body(y_ref, id_ref, w1_ref, g1_ref, b1_ref, w2_ref, g2_ref, b2_ref,
             o_ref):
        yv = y_ref[...]
        m1 = jnp.dot(yv, w1_ref[...], preferred_element_type=jnp.float32)
        mu1 = jnp.mean(m1, axis=0, keepdims=True)
        v1 = jnp.mean((m1 - mu1) * (m1 - mu1), axis=0, keepdims=True)
        h1 = jnp.maximum(
            (m1 - mu1) / jnp.sqrt(v1 + _EPS) * g1_ref[...] + b1_ref[...], 0.0
        )
        m2 = jnp.dot(h1, w2_ref[...], preferred_element_type=jnp.float32)
        mu2 = jnp.mean(m2, axis=0, keepdims=True)
        v2 = jnp.mean((m2 - mu2) * (m2 - mu2), axis=0, keepdims=True)
        h2 = (m2 - mu2) / jnp.sqrt(v2 + _EPS) * g2_ref[...] + b2_ref[...]
        o_ref[...] = jnp.maximum(h2 + id_ref[...], 0.0)

    return pl.pallas_call(
        body,
        out_shape=jax.ShapeDtypeStruct((Q, C), jnp.float32),
    )(y, identity, w1T, g1, b1, w2T, g2, b2)


def _group_conv(q_rows, s_rows, feats, r2, prm):
    B, M, _ = q_rows.shape
    W = prm["W"]
    Cout, Cin = W.shape
    C = Cin - 3
    Wt = 128 * ((Cin + 127) // 128)

    table = jnp.concatenate(
        [s_rows.reshape(-1, 3), feats,
         jnp.zeros((feats.shape[0], Wt - Cin), jnp.float32)], axis=1)
    sT = jnp.transpose(s_rows, (0, 2, 1))
    idx = _knn(q_rows, sT, r2)

    Q = B * M
    qpos = jnp.pad(q_rows.reshape(Q, 3), ((0, 0), (0, 13)))
    rows = _gather_rows(table, idx.reshape(-1), qpos, Q, Wt)

    wT = jnp.pad(jnp.transpose(W), ((0, Wt - Cin), (0, 0)))
    y = _matmul(rows, wT)
    s1 = _colreduce(y)
    s2 = _colreduce(y, s1 / float(Q * _K))
    return _bn_relu_maxpool(
        y.reshape(Q, _K, Cout), s1, s2,
        prm["gamma"].reshape(1, Cout), prm["beta"].reshape(1, Cout),
        float(Q * _K))


def kernel(p0, f0, params):
    B, N, _ = p0.shape
    radii = [0.1, 0.2, 0.4, 0.8]

    f0T = jnp.transpose(f0, (0, 2, 1)).reshape(B * N, -1)
    stem = params["stem"]
    x = _stem_matmul(
        f0T, jnp.transpose(stem["W"]), stem["b"].reshape(1, -1)
    )

    p_rows = p0
    for s in range(4):
        st = params["stages"][s]
        r = radii[s]
        M = N // 4
        q_rows = p_rows[:, ::4, :]
        x = _group_conv(q_rows, p_rows, x, r * r, st["sa"])
        p_rows = q_rows
        N = M
        for blk in st["blocks"]:
            y = _group_conv(p_rows, p_rows, x, 4.0 * r * r, blk["la"])
            x = _pw_block(
                y,
                x,
                jnp.transpose(blk["pw1"]["W"]),
                blk["pw1"]["gamma"].reshape(1, -1),
                blk["pw1"]["beta"].reshape(1, -1),
                jnp.transpose(blk["pw2"]["W"]),
                blk["pw2"]["gamma"].reshape(1, -1),
                blk["pw2"]["beta"].reshape(1, -1),
            )

    C = x.shape[1]
    return jnp.transpose(x.reshape(B, N, C), (0, 2, 1))
